# baseline (speedup 1.0000x reference)
import functools

import jax
import jax.numpy as jnp
from jax import lax
from jax.experimental import pallas as pl
from jax.experimental.pallas import tpu as pltpu
from jax.experimental.pallas import tpu_sc as plsc

D = 128
NF = 4
V = 32
PV = V * V
LANES = 16
NC, NS = 2, 16
NW = NC * NS
NQ = 4
NR = NW // NQ
DQ = D // NQ
T = 256


def _sc_embed(b, l):
    n_tokens = b * l
    tpw = n_tokens // NR
    g_count = tpw // T

    mesh = plsc.VectorSubcoreMesh(
        core_axis_name="c", subcore_axis_name="s",
        num_cores=NC, num_subcores=NS)

    @functools.partial(
        pl.kernel,
        out_type=jax.ShapeDtypeStruct((b * l, D), jnp.float32),
        mesh=mesh,
        compiler_params=pltpu.CompilerParams(use_tc_tiling_on_sc=False),
        scratch_types=[
            pltpu.VMEM((2 * PV, DQ), jnp.float32),
            pltpu.VMEM((2, T), jnp.int32),
            pltpu.VMEM((2, T, DQ), jnp.float32),
            pltpu.SemaphoreType.DMA,
            pltpu.SemaphoreType.DMA,
            pltpu.SemaphoreType.DMA,
            pltpu.SemaphoreType.DMA,
        ],
    )
    def body(x_hbm, w_hbm, o_hbm, w_v, idx_v, out_v, si0, si1, so0, so1):
        sin = (si0, si1)
        sout = (so0, so1)
        wid = lax.axis_index("s") * NC + lax.axis_index("c")
        quarter = wid % NQ
        rng = wid // NQ
        base = rng * tpw
        col0 = quarter * DQ
        o2 = o_hbm

        pltpu.async_copy(x_hbm.at[pl.ds(base, T)], idx_v.at[0], sin[0])
        pltpu.async_copy(x_hbm.at[pl.ds(base + T, T)], idx_v.at[1], sin[1])
        pltpu.sync_copy(w_hbm.at[:, pl.ds(col0, DQ)], w_v)

        @pl.loop(0, g_count, step=2)
        def _(g0):
            for p in range(2):
                g = g0 + p
                start = base + g * T
                pltpu.make_async_copy(
                    x_hbm.at[pl.ds(start, T)], idx_v.at[p], sin[p]).wait()

                @pl.when(g >= 2)
                def _():
                    pltpu.make_async_copy(
                        out_v.at[p],
                        o2.at[pl.ds(start - 2 * T, T), pl.ds(col0, DQ)],
                        sout[p]).wait()

                @plsc.parallel_loop(0, T // 16, unroll=4)
                def _(q):
                    pk = idx_v[p, pl.ds(q * 16, 16)]
                    for k in range(16):
                        t = q * 16 + k
                        w0 = pk[k]
                        i0 = w0 & 0xFFFF
                        i1 = w0 >> 16
                        for j in range(DQ // LANES):
                            sl = pl.ds(j * LANES, LANES)
                            out_v[p, t, sl] = w_v[i0, sl] + w_v[i1, sl]

                pltpu.async_copy(
                    out_v.at[p],
                    o2.at[pl.ds(start, T), pl.ds(col0, DQ)], sout[p])

                @pl.when(g + 2 < g_count)
                def _():
                    pltpu.async_copy(
                        x_hbm.at[pl.ds(start + 2 * T, T)],
                        idx_v.at[p], sin[p])

        for p in range(2):
            pltpu.make_async_copy(
                out_v.at[p],
                o2.at[pl.ds(base + (g_count - 2 + p) * T, T),
                      pl.ds(col0, DQ)],
                sout[p]).wait()

    return body


def kernel(x, W_hour, W_day, W_weekday, W_month):
    b, l, nf = x.shape
    assert nf == NF
    assert (b * l) % (NR * T * 2) == 0
    hd = x[..., 0] * V + x[..., 1]
    wm = x[..., 2] * V + x[..., 3] + PV
    xp = (hd | (wm << 16)).reshape(b * l)
    pair_hd = (W_hour[:, None, :] + W_day[None, :, :]).reshape(PV, D)
    pair_wm = (W_weekday[:, None, :] + W_month[None, :, :]).reshape(PV, D)
    w = jnp.concatenate([pair_hd, pair_wm])
    return _sc_embed(b, l)(xp, w).reshape(b, l, D)


# speedup vs baseline: 1.3573x; 1.3573x over previous
import functools

import jax
import jax.numpy as jnp
from jax import lax
from jax.experimental import pallas as pl
from jax.experimental.pallas import tpu as pltpu
from jax.experimental.pallas import tpu_sc as plsc

D = 128
NF = 4
V = 32
PV = V * V
LANES = 16
NC, NS = 2, 16
NW = NC * NS
NQ = 4
NR = NW // NQ
DQ = D // NQ
T = 512


def _sc_embed(b, l):
    n_tokens = b * l
    tpw = n_tokens // NR
    g_count = tpw // T

    mesh = plsc.VectorSubcoreMesh(
        core_axis_name="c", subcore_axis_name="s",
        num_cores=NC, num_subcores=NS)

    @functools.partial(
        pl.kernel,
        out_type=jax.ShapeDtypeStruct((b * l, D), jnp.float32),
        mesh=mesh,
        compiler_params=pltpu.CompilerParams(use_tc_tiling_on_sc=False),
        scratch_types=[
            pltpu.VMEM((2 * PV, DQ), jnp.float32),
            pltpu.VMEM((2, T), jnp.int32),
            pltpu.VMEM((2, T, DQ), jnp.float32),
            pltpu.SemaphoreType.DMA,
            pltpu.SemaphoreType.DMA,
            pltpu.SemaphoreType.DMA,
            pltpu.SemaphoreType.DMA,
        ],
    )
    def body(x_hbm, w_hbm, o_hbm, w_v, idx_v, out_v, si0, si1, so0, so1):
        sin = (si0, si1)
        sout = (so0, so1)
        wid = lax.axis_index("s") * NC + lax.axis_index("c")
        quarter = wid % NQ
        rng = wid // NQ
        base = rng * tpw
        col0 = quarter * DQ
        o2 = o_hbm

        pltpu.async_copy(x_hbm.at[pl.ds(base, T)], idx_v.at[0], sin[0])
        pltpu.async_copy(x_hbm.at[pl.ds(base + T, T)], idx_v.at[1], sin[1])
        pltpu.sync_copy(w_hbm.at[:, pl.ds(col0, DQ)], w_v)

        @pl.loop(0, g_count, step=2)
        def _(g0):
            for p in range(2):
                g = g0 + p
                start = base + g * T
                pltpu.make_async_copy(
                    x_hbm.at[pl.ds(start, T)], idx_v.at[p], sin[p]).wait()

                @pl.when(g >= 2)
                def _():
                    pltpu.make_async_copy(
                        out_v.at[p],
                        o2.at[pl.ds(start - 2 * T, T), pl.ds(col0, DQ)],
                        sout[p]).wait()

                @plsc.parallel_loop(0, T // 16, unroll=2)
                def _(q):
                    pk = idx_v[p, pl.ds(q * 16, 16)]
                    for k in range(16):
                        t = q * 16 + k
                        w0 = pk[k]
                        i0 = w0 & 0xFFFF
                        i1 = w0 >> 16
                        for j in range(DQ // LANES):
                            sl = pl.ds(j * LANES, LANES)
                            out_v[p, t, sl] = w_v[i0, sl] + w_v[i1, sl]

                pltpu.async_copy(
                    out_v.at[p],
                    o2.at[pl.ds(start, T), pl.ds(col0, DQ)], sout[p])

                @pl.when(g + 2 < g_count)
                def _():
                    pltpu.async_copy(
                        x_hbm.at[pl.ds(start + 2 * T, T)],
                        idx_v.at[p], sin[p])

        for p in range(2):
            pltpu.make_async_copy(
                out_v.at[p],
                o2.at[pl.ds(base + (g_count - 2 + p) * T, T),
                      pl.ds(col0, DQ)],
                sout[p]).wait()

    return body


def kernel(x, W_hour, W_day, W_weekday, W_month):
    b, l, nf = x.shape
    assert nf == NF
    assert (b * l) % (NR * T * 2) == 0
    hd = x[..., 0] * V + x[..., 1]
    wm = x[..., 2] * V + x[..., 3] + PV
    xp = (hd | (wm << 16)).reshape(b * l)
    pair_hd = (W_hour[:, None, :] + W_day[None, :, :]).reshape(PV, D)
    pair_wm = (W_weekday[:, None, :] + W_month[None, :, :]).reshape(PV, D)
    w = jnp.concatenate([pair_hd, pair_wm])
    return _sc_embed(b, l)(xp, w).reshape(b, l, D)


# speedup vs baseline: 1.6053x; 1.1827x over previous
import functools

import jax
import jax.numpy as jnp
from jax import lax
from jax.experimental import pallas as pl
from jax.experimental.pallas import tpu as pltpu
from jax.experimental.pallas import tpu_sc as plsc

D = 128
NF = 4
V = 32
PV = V * V
LANES = 16
NC, NS = 2, 16
NW = NC * NS
NQ = 4
NR = NW // NQ
DQ = D // NQ
T = 800


def _sc_embed(b, l):
    n_tokens = b * l
    tpw = n_tokens // NR
    g_count = tpw // T

    mesh = plsc.VectorSubcoreMesh(
        core_axis_name="c", subcore_axis_name="s",
        num_cores=NC, num_subcores=NS)

    @functools.partial(
        pl.kernel,
        out_type=jax.ShapeDtypeStruct((b * l, D), jnp.float32),
        mesh=mesh,
        compiler_params=pltpu.CompilerParams(use_tc_tiling_on_sc=False),
        scratch_types=[
            pltpu.VMEM((2 * PV, DQ), jnp.float32),
            pltpu.VMEM((2, T), jnp.int32),
            pltpu.VMEM((2, T, DQ), jnp.float32),
            pltpu.SemaphoreType.DMA,
            pltpu.SemaphoreType.DMA,
            pltpu.SemaphoreType.DMA,
            pltpu.SemaphoreType.DMA,
        ],
    )
    def body(x_hbm, w_hbm, o_hbm, w_v, idx_v, out_v, si0, si1, so0, so1):
        sin = (si0, si1)
        sout = (so0, so1)
        wid = lax.axis_index("s") * NC + lax.axis_index("c")
        quarter = wid % NQ
        rng = wid // NQ
        base = rng * tpw
        col0 = quarter * DQ
        o2 = o_hbm

        pltpu.async_copy(x_hbm.at[pl.ds(base, T)], idx_v.at[0], sin[0])
        pltpu.async_copy(x_hbm.at[pl.ds(base + T, T)], idx_v.at[1], sin[1])
        pltpu.sync_copy(w_hbm.at[:, pl.ds(col0, DQ)], w_v)

        @pl.loop(0, g_count, step=2)
        def _(g0):
            for p in range(2):
                g = g0 + p
                start = base + g * T
                pltpu.make_async_copy(
                    x_hbm.at[pl.ds(start, T)], idx_v.at[p], sin[p]).wait()

                @pl.when(g >= 2)
                def _():
                    pltpu.make_async_copy(
                        out_v.at[p],
                        o2.at[pl.ds(start - 2 * T, T), pl.ds(col0, DQ)],
                        sout[p]).wait()

                @plsc.parallel_loop(0, T // 16, unroll=2)
                def _(q):
                    pk = idx_v[p, pl.ds(q * 16, 16)]
                    for k in range(16):
                        t = q * 16 + k
                        w0 = pk[k]
                        i0 = w0 & 0xFFFF
                        i1 = w0 >> 16
                        for j in range(DQ // LANES):
                            sl = pl.ds(j * LANES, LANES)
                            out_v[p, t, sl] = w_v[i0, sl] + w_v[i1, sl]

                pltpu.async_copy(
                    out_v.at[p],
                    o2.at[pl.ds(start, T), pl.ds(col0, DQ)], sout[p])

                @pl.when(g + 2 < g_count)
                def _():
                    pltpu.async_copy(
                        x_hbm.at[pl.ds(start + 2 * T, T)],
                        idx_v.at[p], sin[p])

        for p in range(2):
            pltpu.make_async_copy(
                out_v.at[p],
                o2.at[pl.ds(base + (g_count - 2 + p) * T, T),
                      pl.ds(col0, DQ)],
                sout[p]).wait()

    return body


def kernel(x, W_hour, W_day, W_weekday, W_month):
    b, l, nf = x.shape
    assert nf == NF
    assert (b * l) % (NR * T * 2) == 0
    hd = x[..., 0] * V + x[..., 1]
    wm = x[..., 2] * V + x[..., 3] + PV
    xp = (hd | (wm << 16)).reshape(b * l)
    pair_hd = (W_hour[:, None, :] + W_day[None, :, :]).reshape(PV, D)
    pair_wm = (W_weekday[:, None, :] + W_month[None, :, :]).reshape(PV, D)
    w = jnp.concatenate([pair_hd, pair_wm])
    return _sc_embed(b, l)(xp, w).reshape(b, l, D)
